# baseline (speedup 1.0000x reference)
import functools

import jax
import jax.numpy as jnp
from jax import lax
from jax.experimental import pallas as pl
from jax.experimental.pallas import tpu as pltpu
from jax.experimental.pallas import tpu_sc as plsc

NC, NS, LANES = 2, 16, 16
NW = NC * NS
CH = 80


def _node_tables_body(z_ref, w1ab_ref, w1v_ref, zv_ref, b1_ref,
                      ab_ref, c_ref):
    ab_ref[...] = jnp.dot(z_ref[...], w1ab_ref[...],
                          preferred_element_type=jnp.float32)
    c_ref[...] = (jnp.dot(zv_ref[...], w1v_ref[...],
                          preferred_element_type=jnp.float32) + b1_ref[...])


def _sc_gather_body(ab_hbm, idxsd_hbm, out_hbm,
                    idx, rows_s, rows_t, rows_o, tab,
                    sem_s0, sem_s1, sem_t0, sem_t1, sem_o, sem_i0, sem_i1,
                    n_chunks):
    cid = lax.axis_index("c")
    sid = lax.axis_index("s")
    wid = sid * NC + cid
    half = CH // 2
    sem_s = (sem_s0, sem_s1)
    sem_t = (sem_t0, sem_t1)
    sem_i = (sem_i0, sem_i1)
    base = wid * n_chunks

    n_nodes = ab_hbm.shape[0]
    rows_per_stager = n_nodes // 10

    @pl.when(sid < 10)
    def _stage():
        pltpu.sync_copy(ab_hbm.at[pl.ds(sid * rows_per_stager,
                                        rows_per_stager)],
                        tab.at[pl.ds(sid * rows_per_stager, rows_per_stager)])
    plsc.subcore_barrier()

    def start_idx(i, b):
        pltpu.async_copy(idxsd_hbm.at[wid, i], idx.at[b], sem_i[b])

    def wait_idx(b):
        pltpu.make_async_copy(idxsd_hbm.at[wid, 0], idx.at[b],
                              sem_i[b]).wait()

    def start_gather(b):
        src = tab if b == 0 else ab_hbm
        pltpu.async_copy(src.at[idx.at[b, 0]], rows_s.at[b], sem_s[b])
        pltpu.async_copy(src.at[idx.at[b, 1]], rows_t.at[b], sem_t[b])

    def wait_gather(b):
        src = tab if b == 0 else ab_hbm
        pltpu.make_async_copy(src.at[idx.at[b, 0]], rows_s.at[b],
                              sem_s[b]).wait()
        pltpu.make_async_copy(src.at[idx.at[b, 1]], rows_t.at[b],
                              sem_t[b]).wait()

    def start_write(i):
        pltpu.async_copy(rows_o,
                         out_hbm.at[pl.ds((base + i) * half, half)], sem_o)

    def wait_write():
        pltpu.make_async_copy(rows_o, out_hbm.at[pl.ds(0, half)],
                              sem_o).wait()

    def add_pack(b):
        @pl.loop(0, half)
        def _row(r):
            for c in range(8):
                e = (c // 4) * half + r
                col = (c % 4) * LANES
                rows_o[r, pl.ds(c * LANES, LANES)] = (
                    rows_s[b, e, pl.ds(col, LANES)]
                    + rows_t[b, e, pl.ds(64 + col, LANES)])

    assert n_chunks % 2 == 1 and n_chunks >= 3
    start_idx(0, 0)
    start_idx(1, 1)
    wait_idx(0)
    start_gather(0)
    wait_idx(1)
    start_gather(1)
    wait_gather(0)
    start_idx(2, 0)
    add_pack(0)
    start_write(0)
    wait_idx(0)
    start_gather(0)
    wait_gather(1)
    start_idx(3, 1)
    wait_write()
    add_pack(1)
    start_write(1)

    @pl.loop(1, (n_chunks - 1) // 2)
    def _pair(p):
        i0 = 2 * p
        for b in range(2):
            i = i0 + b
            wait_idx(1 - b)
            start_gather(1 - b)
            wait_gather(b)

            @pl.when(i + 2 < n_chunks)
            def _refill():
                start_idx(i + 2, b)
            wait_write()
            add_pack(b)
            start_write(i)

    i_last = n_chunks - 1
    wait_gather(0)
    wait_write()
    add_pack(0)
    start_write(i_last)
    wait_write()


def _mlp_body(g_ref, c_ref, w2_ref, b2_ref, w3t_ref, b3_ref, o_ref):
    h1 = jnp.maximum(g_ref[...] + c_ref[...], 0.0)
    h2 = jnp.maximum(
        jnp.dot(h1, w2_ref[...], preferred_element_type=jnp.float32)
        + b2_ref[...], 0.0)
    om_t = lax.dot_general(w3t_ref[...], h2, (((1,), (1,)), ((), ())),
                           preferred_element_type=jnp.float32)
    o_ref[...] = jax.nn.sigmoid(om_t + b3_ref[...])


def kernel(Z, edge_index, node_idx, W1, b1, W2, b2, W3, b3):
    N, D = Z.shape
    E = edge_index.shape[1]
    H1 = W1.shape[1]
    H2 = W2.shape[1]

    W1ab = jnp.concatenate([W1[:D], W1[D:2 * D]], axis=1)
    W1v = W1[2 * D:]
    zv = lax.dynamic_slice_in_dim(Z, node_idx, 1, axis=0)

    AB, c = pl.pallas_call(
        _node_tables_body,
        out_shape=(
            jax.ShapeDtypeStruct((N, 2 * H1), jnp.float32),
            jax.ShapeDtypeStruct((1, H1), jnp.float32),
        ),
    )(Z, W1ab, W1v, zv, b1.reshape(1, H1))

    per_w = E // NW
    n_chunks = per_w // CH
    assert per_w % CH == 0
    half = CH // 2
    srci = jnp.concatenate(
        [edge_index[0, :E // 2].reshape(NW, n_chunks, half),
         edge_index[0, E // 2:].reshape(NW, n_chunks, half)], axis=2)
    dsti = jnp.concatenate(
        [edge_index[1, :E // 2].reshape(NW, n_chunks, half),
         edge_index[1, E // 2:].reshape(NW, n_chunks, half)], axis=2)
    idxsd = jnp.stack([srci, dsti], axis=2)

    mesh = plsc.VectorSubcoreMesh(core_axis_name="c", subcore_axis_name="s",
                                  num_cores=NC, num_subcores=NS)
    sc_fn = pl.kernel(
        functools.partial(_sc_gather_body, n_chunks=n_chunks),
        out_type=jax.ShapeDtypeStruct((E // 2, 2 * H1), jnp.float32),
        mesh=mesh,
        scratch_types=[
            pltpu.VMEM((2, 2, CH), jnp.int32),
            pltpu.VMEM((2, CH, 2 * H1), jnp.float32),
            pltpu.VMEM((2, CH, 2 * H1), jnp.float32),
            pltpu.VMEM((CH // 2, 2 * H1), jnp.float32),
            pltpu.VMEM_SHARED((N, 2 * H1), jnp.float32),
            pltpu.SemaphoreType.DMA,
            pltpu.SemaphoreType.DMA,
            pltpu.SemaphoreType.DMA,
            pltpu.SemaphoreType.DMA,
            pltpu.SemaphoreType.DMA,
            pltpu.SemaphoreType.DMA,
            pltpu.SemaphoreType.DMA,
        ],
    )
    G = sc_fn(AB, idxsd)

    cc = jnp.concatenate([c, c], axis=1)
    W2p = jnp.zeros((2 * H1, 2 * H2), jnp.float32)
    W2p = W2p.at[:H1, :H2].set(W2).at[H1:, H2:].set(W2)
    b2p = jnp.concatenate([b2, b2]).reshape(1, 2 * H2)
    W3t = jnp.zeros((2, 2 * H2), jnp.float32)
    W3t = W3t.at[0, :H2].set(W3[:, 0]).at[1, H2:].set(W3[:, 0])
    b3t = jnp.broadcast_to(b3.reshape(1, 1), (2, 1))

    BR = 3200
    out2 = pl.pallas_call(
        _mlp_body,
        grid=(E // 2 // BR,),
        in_specs=[
            pl.BlockSpec((BR, 2 * H1), lambda i: (i, 0)),
            pl.BlockSpec((1, 2 * H1), lambda i: (0, 0)),
            pl.BlockSpec((2 * H1, 2 * H2), lambda i: (0, 0)),
            pl.BlockSpec((1, 2 * H2), lambda i: (0, 0)),
            pl.BlockSpec((2, 2 * H2), lambda i: (0, 0)),
            pl.BlockSpec((2, 1), lambda i: (0, 0)),
        ],
        out_specs=pl.BlockSpec((2, BR), lambda i: (0, i)),
        out_shape=jax.ShapeDtypeStruct((2, E // 2), jnp.float32),
    )(G, cc, W2p, b2p, W3t, b3t)

    return out2.reshape(E)


# speedup vs baseline: 1.1430x; 1.1430x over previous
import functools

import jax
import jax.numpy as jnp
from jax import lax
from jax.experimental import pallas as pl
from jax.experimental.pallas import tpu as pltpu
from jax.experimental.pallas import tpu_sc as plsc

NC, NS, LANES = 2, 16, 16
NW = NC * NS
CH = 80


def _node_tables_body(z_ref, w1ab_ref, w1v_ref, zv_ref, b1_ref,
                      ab_ref, c_ref):
    ab_ref[...] = jnp.dot(z_ref[...], w1ab_ref[...],
                          preferred_element_type=jnp.float32)
    c_ref[...] = (jnp.dot(zv_ref[...], w1v_ref[...],
                          preferred_element_type=jnp.float32) + b1_ref[...])


def _sc_gather_body(ab_hbm, idxsd_hbm, out_hbm,
                    idx, rows_s, rows_t, rows_o, tab,
                    sem_s0, sem_s1, sem_t0, sem_t1, sem_o, sem_i0, sem_i1,
                    n_chunks):
    cid = lax.axis_index("c")
    sid = lax.axis_index("s")
    wid = sid * NC + cid
    half = CH // 2
    sem_s = (sem_s0, sem_s1)
    sem_t = (sem_t0, sem_t1)
    sem_i = (sem_i0, sem_i1)
    base = wid * n_chunks

    n_nodes = ab_hbm.shape[0]
    rows_per_stager = n_nodes // 10

    @pl.when(sid < 10)
    def _stage():
        pltpu.sync_copy(ab_hbm.at[pl.ds(sid * rows_per_stager,
                                        rows_per_stager)],
                        tab.at[pl.ds(sid * rows_per_stager, rows_per_stager)])
    plsc.subcore_barrier()

    def start_idx(i, b):
        pltpu.async_copy(idxsd_hbm.at[wid, i], idx.at[b], sem_i[b])

    def wait_idx(b):
        pltpu.make_async_copy(idxsd_hbm.at[wid, 0], idx.at[b],
                              sem_i[b]).wait()

    def start_gather(b):
        pltpu.async_copy(tab.at[idx.at[b, 0]], rows_s.at[b], sem_s[b])
        pltpu.async_copy(tab.at[idx.at[b, 1]], rows_t.at[b], sem_t[b])

    def wait_gather(b):
        pltpu.make_async_copy(tab.at[idx.at[b, 0]], rows_s.at[b],
                              sem_s[b]).wait()
        pltpu.make_async_copy(tab.at[idx.at[b, 1]], rows_t.at[b],
                              sem_t[b]).wait()

    def start_write(i):
        pltpu.async_copy(rows_o,
                         out_hbm.at[pl.ds((base + i) * half, half)], sem_o)

    def wait_write():
        pltpu.make_async_copy(rows_o, out_hbm.at[pl.ds(0, half)],
                              sem_o).wait()

    def add_pack(b):
        @pl.loop(0, half)
        def _row(r):
            for c in range(8):
                e = (c // 4) * half + r
                col = (c % 4) * LANES
                rows_o[r, pl.ds(c * LANES, LANES)] = (
                    rows_s[b, e, pl.ds(col, LANES)]
                    + rows_t[b, e, pl.ds(64 + col, LANES)])

    assert n_chunks % 2 == 1 and n_chunks >= 3
    start_idx(0, 0)
    start_idx(1, 1)
    wait_idx(0)
    start_gather(0)
    wait_idx(1)
    start_gather(1)
    wait_gather(0)
    start_idx(2, 0)
    add_pack(0)
    start_write(0)
    wait_idx(0)
    start_gather(0)
    wait_gather(1)
    start_idx(3, 1)
    wait_write()
    add_pack(1)
    start_write(1)

    @pl.loop(1, (n_chunks - 1) // 2)
    def _pair(p):
        i0 = 2 * p
        for b in range(2):
            i = i0 + b
            wait_idx(1 - b)
            start_gather(1 - b)
            wait_gather(b)

            @pl.when(i + 2 < n_chunks)
            def _refill():
                start_idx(i + 2, b)
            wait_write()
            add_pack(b)
            start_write(i)

    i_last = n_chunks - 1
    wait_gather(0)
    wait_write()
    add_pack(0)
    start_write(i_last)
    wait_write()


def _mlp_body(g_ref, c_ref, w2_ref, b2_ref, w3t_ref, b3_ref, o_ref):
    h1 = jnp.maximum(g_ref[...] + c_ref[...], 0.0)
    h2 = jnp.maximum(
        jnp.dot(h1, w2_ref[...], preferred_element_type=jnp.float32)
        + b2_ref[...], 0.0)
    om_t = lax.dot_general(w3t_ref[...], h2, (((1,), (1,)), ((), ())),
                           preferred_element_type=jnp.float32)
    o_ref[...] = jax.nn.sigmoid(om_t + b3_ref[...])


def kernel(Z, edge_index, node_idx, W1, b1, W2, b2, W3, b3):
    N, D = Z.shape
    E = edge_index.shape[1]
    H1 = W1.shape[1]
    H2 = W2.shape[1]

    W1ab = jnp.concatenate([W1[:D], W1[D:2 * D]], axis=1)
    W1v = W1[2 * D:]
    zv = lax.dynamic_slice_in_dim(Z, node_idx, 1, axis=0)

    AB, c = pl.pallas_call(
        _node_tables_body,
        out_shape=(
            jax.ShapeDtypeStruct((N, 2 * H1), jnp.float32),
            jax.ShapeDtypeStruct((1, H1), jnp.float32),
        ),
    )(Z, W1ab, W1v, zv, b1.reshape(1, H1))

    per_w = E // NW
    n_chunks = per_w // CH
    assert per_w % CH == 0
    half = CH // 2
    srci = jnp.concatenate(
        [edge_index[0, :E // 2].reshape(NW, n_chunks, half),
         edge_index[0, E // 2:].reshape(NW, n_chunks, half)], axis=2)
    dsti = jnp.concatenate(
        [edge_index[1, :E // 2].reshape(NW, n_chunks, half),
         edge_index[1, E // 2:].reshape(NW, n_chunks, half)], axis=2)
    idxsd = jnp.stack([srci, dsti], axis=2)

    mesh = plsc.VectorSubcoreMesh(core_axis_name="c", subcore_axis_name="s",
                                  num_cores=NC, num_subcores=NS)
    sc_fn = pl.kernel(
        functools.partial(_sc_gather_body, n_chunks=n_chunks),
        out_type=jax.ShapeDtypeStruct((E // 2, 2 * H1), jnp.float32),
        mesh=mesh,
        scratch_types=[
            pltpu.VMEM((2, 2, CH), jnp.int32),
            pltpu.VMEM((2, CH, 2 * H1), jnp.float32),
            pltpu.VMEM((2, CH, 2 * H1), jnp.float32),
            pltpu.VMEM((CH // 2, 2 * H1), jnp.float32),
            pltpu.VMEM_SHARED((N, 2 * H1), jnp.float32),
            pltpu.SemaphoreType.DMA,
            pltpu.SemaphoreType.DMA,
            pltpu.SemaphoreType.DMA,
            pltpu.SemaphoreType.DMA,
            pltpu.SemaphoreType.DMA,
            pltpu.SemaphoreType.DMA,
            pltpu.SemaphoreType.DMA,
        ],
    )
    G = sc_fn(AB, idxsd)

    cc = jnp.concatenate([c, c], axis=1)
    W2p = jnp.zeros((2 * H1, 2 * H2), jnp.float32)
    W2p = W2p.at[:H1, :H2].set(W2).at[H1:, H2:].set(W2)
    b2p = jnp.concatenate([b2, b2]).reshape(1, 2 * H2)
    W3t = jnp.zeros((2, 2 * H2), jnp.float32)
    W3t = W3t.at[0, :H2].set(W3[:, 0]).at[1, H2:].set(W3[:, 0])
    b3t = jnp.broadcast_to(b3.reshape(1, 1), (2, 1))

    BR = 3200
    out2 = pl.pallas_call(
        _mlp_body,
        grid=(E // 2 // BR,),
        in_specs=[
            pl.BlockSpec((BR, 2 * H1), lambda i: (i, 0)),
            pl.BlockSpec((1, 2 * H1), lambda i: (0, 0)),
            pl.BlockSpec((2 * H1, 2 * H2), lambda i: (0, 0)),
            pl.BlockSpec((1, 2 * H2), lambda i: (0, 0)),
            pl.BlockSpec((2, 2 * H2), lambda i: (0, 0)),
            pl.BlockSpec((2, 1), lambda i: (0, 0)),
        ],
        out_specs=pl.BlockSpec((2, BR), lambda i: (0, i)),
        out_shape=jax.ShapeDtypeStruct((2, E // 2), jnp.float32),
    )(G, cc, W2p, b2p, W3t, b3t)

    return out2.reshape(E)
